# baseline (speedup 1.0000x reference)
import jax
import jax.numpy as jnp
from jax import lax
from jax.experimental import pallas as pl
from jax.experimental.pallas import tpu as pltpu
from jax.experimental.pallas import tpu_sc as plsc

N = 10000
K = 5000
D = 128
KPAD = 5120
NFULL = 312
TAIL = K - 16
GR = 4
NG = K // GR
G_SMALL = 39
W_BIG = 2
MAXPAIRS = 20


def _sc_gather_body(a_hbm, h_hbm, idx_hbm, idx2d_hbm, out_a, out_h,
                    colidx_v, idx2d_v, rb0, rb1, ob0, ob1, hb0, hb1,
                    si0, si1, shi0, shi1, so0, so1, sho0, sho1):
    cid = lax.axis_index("c")
    sid = lax.axis_index("s")
    w = sid * 2 + cid
    b0 = w * G_SMALL + jnp.minimum(w, W_BIG)
    nb = G_SMALL + (w < W_BIG).astype(jnp.int32)

    pltpu.sync_copy(idx_hbm, colidx_v)
    pltpu.sync_copy(idx2d_hbm.at[w], idx2d_v)

    def start_in(g, rb, hb, sa, sh):
        idx4 = idx2d_v.at[g]
        pltpu.async_copy(a_hbm.at[idx4], rb, sa)
        pltpu.async_copy(h_hbm.at[idx4], hb, sh)

    def wait_in(rb, hb, sa, sh):
        pltpu.make_async_copy(a_hbm.at[pl.ds(0, GR)], rb, sa).wait()
        pltpu.make_async_copy(h_hbm.at[pl.ds(0, GR)], hb, sh).wait()

    def wait_out(ob, hb, sa, sh):
        pltpu.make_async_copy(ob, out_a.at[pl.ds(0, GR)], sa).wait()
        pltpu.make_async_copy(hb, out_h.at[pl.ds(0, GR)], sh).wait()

    def compute(rb, ob):
        for r in range(GR):
            rsp = lax.full((16,), r, jnp.int32)

            def chunk_body(j, c, r=r, rsp=rsp):
                cidx = colidx_v[pl.ds(j * 16, 16)]
                ob[r, pl.ds(j * 16, 16)] = plsc.load_gather(rb, [rsp, cidx])
                return c

            lax.fori_loop(0, NFULL, chunk_body, 0, unroll=8)
            cidx = colidx_v[pl.ds(TAIL, 16)]
            ob[r, pl.ds(TAIL, 16)] = plsc.load_gather(rb, [rsp, cidx])

    def start_out(g, ob, hb, sa, sh):
        rg = (b0 + g) * GR
        pltpu.async_copy(ob, out_a.at[pl.ds(rg, GR)], sa)
        pltpu.async_copy(hb, out_h.at[pl.ds(rg, GR)], sh)

    @pl.when(nb > 0)
    def _prologue():
        start_in(0, rb0, hb0, si0, shi0)

    def pair_body(i, carry):
        ge = 2 * i
        go = 2 * i + 1

        @pl.when(ge < nb)
        def _even():
            wait_in(rb0, hb0, si0, shi0)

            @pl.when(go < nb)
            def _():
                start_in(go, rb1, hb1, si1, shi1)

            @pl.when(i > 0)
            def _():
                wait_out(ob0, hb0, so0, sho0)

            compute(rb0, ob0)
            start_out(ge, ob0, hb0, so0, sho0)

        @pl.when(go < nb)
        def _odd():
            wait_in(rb1, hb1, si1, shi1)

            @pl.when(go + 1 < nb)
            def _():
                start_in(go + 1, rb0, hb0, si0, shi0)

            @pl.when(i > 0)
            def _():
                wait_out(ob1, hb1, so1, sho1)

            compute(rb1, ob1)
            start_out(go, ob1, hb1, so1, sho1)

        return carry

    lax.fori_loop(0, MAXPAIRS, pair_body, 0)

    @pl.when(nb > 0)
    def _drain0():
        wait_out(ob0, hb0, so0, sho0)

    @pl.when(nb > 1)
    def _drain1():
        wait_out(ob1, hb1, so1, sho1)


def _scale_body(h_ref, v_ref, o_ref):
    o_ref[...] = h_ref[...] * v_ref[...]


def _scale_rows(h_raw, values):
    return pl.pallas_call(
        _scale_body,
        grid=(5,),
        in_specs=[
            pl.BlockSpec((K // 5, D), lambda i: (i, 0)),
            pl.BlockSpec((K // 5, 1), lambda i: (i, 0)),
        ],
        out_specs=pl.BlockSpec((K // 5, D), lambda i: (i, 0)),
        out_shape=jax.ShapeDtypeStruct((K, D), jnp.float32),
    )(h_raw, values.reshape(K, 1))


def kernel(scores, H, A, pooling_size):
    del pooling_size
    values, idx = lax.top_k(scores, K)
    idx = idx.astype(jnp.int32)
    b0s = jnp.array([wi * G_SMALL + min(wi, W_BIG) for wi in range(32)],
                    jnp.int32)
    gidx = jnp.clip(b0s[:, None] + jnp.arange(2 * MAXPAIRS,
                                              dtype=jnp.int32)[None, :],
                    0, NG - 1)
    idx2d = idx.reshape(NG, GR)[gidx]

    mesh = plsc.VectorSubcoreMesh(core_axis_name="c", subcore_axis_name="s")
    pooled_A, h_raw = pl.kernel(
        _sc_gather_body,
        out_type=[
            jax.ShapeDtypeStruct((K, K), jnp.float32),
            jax.ShapeDtypeStruct((K, D), jnp.float32),
        ],
        mesh=mesh,
        compiler_params=pltpu.CompilerParams(
            needs_layout_passes=False, use_tc_tiling_on_sc=False),
        scratch_types=[
            pltpu.VMEM((K,), jnp.int32),
            pltpu.VMEM((2 * MAXPAIRS, GR), jnp.int32),
            pltpu.VMEM((GR, N), jnp.float32),
            pltpu.VMEM((GR, N), jnp.float32),
            pltpu.VMEM((GR, K), jnp.float32),
            pltpu.VMEM((GR, K), jnp.float32),
            pltpu.VMEM((GR, D), jnp.float32),
            pltpu.VMEM((GR, D), jnp.float32),
            pltpu.SemaphoreType.DMA,
            pltpu.SemaphoreType.DMA,
            pltpu.SemaphoreType.DMA,
            pltpu.SemaphoreType.DMA,
            pltpu.SemaphoreType.DMA,
            pltpu.SemaphoreType.DMA,
            pltpu.SemaphoreType.DMA,
            pltpu.SemaphoreType.DMA,
        ],
    )(A, H, idx, idx2d)
    pooled_H = _scale_rows(h_raw, values)
    return (pooled_H, pooled_A, idx)


# speedup vs baseline: 1.6280x; 1.6280x over previous
import jax
import jax.numpy as jnp
from jax import lax
from jax.experimental import pallas as pl
from jax.experimental.pallas import tpu as pltpu
from jax.experimental.pallas import tpu_sc as plsc

N = 10000
K = 5000
D = 128
KPAD = 5120
NFULL = 312
TAIL = K - 16
GR = 4
NG = K // GR
G_SMALL = 39
W_BIG = 2
MAXPAIRS = 20


def _sc_gather_body(a_hbm, h_hbm, idx_hbm, idx2d_hbm, out_a, out_h,
                    colidx_v, idx2d_v, rb0, rb1, ob0, ob1, hb0, hb1,
                    si0, si1, shi0, shi1, so0, so1, sho0, sho1):
    cid = lax.axis_index("c")
    sid = lax.axis_index("s")
    w = sid * 2 + cid
    b0 = w * G_SMALL + jnp.minimum(w, W_BIG)
    nb = G_SMALL + (w < W_BIG).astype(jnp.int32)

    pltpu.sync_copy(idx_hbm, colidx_v)
    pltpu.sync_copy(idx2d_hbm.at[w], idx2d_v)

    def start_in(g, rb, hb, sa, sh):
        idx4 = idx2d_v.at[g]
        pltpu.async_copy(a_hbm.at[idx4], rb, sa)
        pltpu.async_copy(h_hbm.at[idx4], hb, sh)

    def wait_in(rb, hb, sa, sh):
        pltpu.make_async_copy(a_hbm.at[pl.ds(0, GR)], rb, sa).wait()
        pltpu.make_async_copy(h_hbm.at[pl.ds(0, GR)], hb, sh).wait()

    def wait_out(ob, hb, sa, sh):
        pltpu.make_async_copy(ob, out_a.at[pl.ds(0, GR)], sa).wait()
        pltpu.make_async_copy(hb, out_h.at[pl.ds(0, GR)], sh).wait()

    rsps = [lax.full((16,), r, jnp.int32) for r in range(GR)]

    def compute(rb, ob):
        @plsc.parallel_loop(0, NFULL, 1, unroll=8)
        def _(j):
            cidx = colidx_v[pl.ds(j * 16, 16)]
            for r in range(GR):
                ob[r, pl.ds(j * 16, 16)] = plsc.load_gather(rb, [rsps[r], cidx])

        cidx = colidx_v[pl.ds(TAIL, 16)]
        for r in range(GR):
            ob[r, pl.ds(TAIL, 16)] = plsc.load_gather(rb, [rsps[r], cidx])

    def start_out(g, ob, hb, sa, sh):
        rg = (b0 + g) * GR
        pltpu.async_copy(ob, out_a.at[pl.ds(rg, GR)], sa)
        pltpu.async_copy(hb, out_h.at[pl.ds(rg, GR)], sh)

    @pl.when(nb > 0)
    def _prologue():
        start_in(0, rb0, hb0, si0, shi0)

    def pair_body(i, carry):
        ge = 2 * i
        go = 2 * i + 1

        @pl.when(ge < nb)
        def _even():
            wait_in(rb0, hb0, si0, shi0)

            @pl.when(go < nb)
            def _():
                start_in(go, rb1, hb1, si1, shi1)

            @pl.when(i > 0)
            def _():
                wait_out(ob0, hb0, so0, sho0)

            compute(rb0, ob0)
            start_out(ge, ob0, hb0, so0, sho0)

        @pl.when(go < nb)
        def _odd():
            wait_in(rb1, hb1, si1, shi1)

            @pl.when(go + 1 < nb)
            def _():
                start_in(go + 1, rb0, hb0, si0, shi0)

            @pl.when(i > 0)
            def _():
                wait_out(ob1, hb1, so1, sho1)

            compute(rb1, ob1)
            start_out(go, ob1, hb1, so1, sho1)

        return carry

    lax.fori_loop(0, MAXPAIRS, pair_body, 0)

    @pl.when(nb > 0)
    def _drain0():
        wait_out(ob0, hb0, so0, sho0)

    @pl.when(nb > 1)
    def _drain1():
        wait_out(ob1, hb1, so1, sho1)


def _scale_body(h_ref, v_ref, o_ref):
    o_ref[...] = h_ref[...] * v_ref[...]


def _scale_rows(h_raw, values):
    return pl.pallas_call(
        _scale_body,
        grid=(5,),
        in_specs=[
            pl.BlockSpec((K // 5, D), lambda i: (i, 0)),
            pl.BlockSpec((K // 5, 1), lambda i: (i, 0)),
        ],
        out_specs=pl.BlockSpec((K // 5, D), lambda i: (i, 0)),
        out_shape=jax.ShapeDtypeStruct((K, D), jnp.float32),
    )(h_raw, values.reshape(K, 1))


def kernel(scores, H, A, pooling_size):
    del pooling_size
    values, idx = lax.top_k(scores, K)
    idx = idx.astype(jnp.int32)
    b0s = jnp.array([wi * G_SMALL + min(wi, W_BIG) for wi in range(32)],
                    jnp.int32)
    gidx = jnp.clip(b0s[:, None] + jnp.arange(2 * MAXPAIRS,
                                              dtype=jnp.int32)[None, :],
                    0, NG - 1)
    idx2d = idx.reshape(NG, GR)[gidx]

    mesh = plsc.VectorSubcoreMesh(core_axis_name="c", subcore_axis_name="s")
    pooled_A, h_raw = pl.kernel(
        _sc_gather_body,
        out_type=[
            jax.ShapeDtypeStruct((K, K), jnp.float32),
            jax.ShapeDtypeStruct((K, D), jnp.float32),
        ],
        mesh=mesh,
        compiler_params=pltpu.CompilerParams(
            needs_layout_passes=False, use_tc_tiling_on_sc=False),
        scratch_types=[
            pltpu.VMEM((K,), jnp.int32),
            pltpu.VMEM((2 * MAXPAIRS, GR), jnp.int32),
            pltpu.VMEM((GR, N), jnp.float32),
            pltpu.VMEM((GR, N), jnp.float32),
            pltpu.VMEM((GR, K), jnp.float32),
            pltpu.VMEM((GR, K), jnp.float32),
            pltpu.VMEM((GR, D), jnp.float32),
            pltpu.VMEM((GR, D), jnp.float32),
            pltpu.SemaphoreType.DMA,
            pltpu.SemaphoreType.DMA,
            pltpu.SemaphoreType.DMA,
            pltpu.SemaphoreType.DMA,
            pltpu.SemaphoreType.DMA,
            pltpu.SemaphoreType.DMA,
            pltpu.SemaphoreType.DMA,
            pltpu.SemaphoreType.DMA,
        ],
    )(A, H, idx, idx2d)
    pooled_H = _scale_rows(h_raw, values)
    return (pooled_H, pooled_A, idx)
